# baseline (speedup 1.0000x reference)
import functools

import jax
import jax.numpy as jnp
from jax import lax
from jax.experimental import pallas as pl
from jax.experimental.pallas import tpu as pltpu
from jax.experimental.pallas import tpu_sc as plsc

_NC = 2
_NS = 16
_NW = _NC * _NS
_NBUF = 2


@functools.partial(jax.jit, static_argnums=(2, 3, 4))
def _sc_gather(table, idx, b_per_w, chunk, n_outer):
    V, D = table.shape
    B = idx.shape[0]
    mesh = plsc.VectorSubcoreMesh(core_axis_name="c", subcore_axis_name="s")

    @functools.partial(
        pl.kernel,
        out_type=jax.ShapeDtypeStruct((B, D), jnp.float32),
        mesh=mesh,
        scratch_types=[
            pltpu.VMEM((b_per_w,), jnp.int32),
            [pltpu.VMEM((chunk, D), jnp.float32) for _ in range(_NBUF)],
            [pltpu.SemaphoreType.DMA for _ in range(_NBUF)],
            [pltpu.SemaphoreType.DMA for _ in range(_NBUF)],
        ],
    )
    def k(table_hbm, idx_hbm, out_hbm, idx_v, bufs, semg, semw):
        wid = lax.axis_index("s") * _NC + lax.axis_index("c")
        base = wid * b_per_w
        pltpu.sync_copy(idx_hbm.at[pl.ds(base, b_per_w)], idx_v)

        def gather_start(g, b):
            off = g * chunk
            for q in range(chunk // 16):
                vec = idx_v[pl.ds(off + q * 16, 16)]
                for r in range(16):
                    pltpu.make_async_copy(
                        table_hbm.at[pl.ds(vec[r], 1), :],
                        bufs[b].at[pl.ds(q * 16 + r, 1), :],
                        semg[b],
                    ).start()

        def gather_wait(b):
            pltpu.make_async_copy(
                table_hbm.at[pl.ds(0, chunk), :], bufs[b], semg[b]
            ).wait()

        def write_start(g, b):
            off = g * chunk
            pltpu.make_async_copy(
                bufs[b], out_hbm.at[pl.ds(base + off, chunk)], semw[b]
            ).start()

        def write_wait(b):
            pltpu.make_async_copy(
                bufs[b], out_hbm.at[pl.ds(base, chunk)], semw[b]
            ).wait()

        for b in range(_NBUF):
            gather_start(b, b)

        def body(j, _):
            for b in range(_NBUF):
                gather_wait(b)
                write_start(j * _NBUF + b, b)

            for b in range(_NBUF):
                write_wait(b)

                @pl.when(j < n_outer - 1)
                def _():
                    gather_start((j + 1) * _NBUF + b, b)

            return ()

        lax.fori_loop(0, n_outer, body, ())

    return k(table, idx)


def _fmt_body(y_ref, o_ref):
    x = y_ref[...]
    for s in range(y_ref.shape[1]):
        o_ref[s] = x[:, s, :].T


@functools.partial(jax.jit, static_argnums=(1, 2, 3, 4))
def _tc_format(y3, Bx, S, D, bb):
    sb = 8
    fmt = pl.pallas_call(
        _fmt_body,
        grid=(Bx // bb, S // sb),
        in_specs=[
            pl.BlockSpec((bb, sb, D), lambda bi, si: (bi, si, 0)),
        ],
        out_specs=pl.BlockSpec((sb, D, bb), lambda bi, si: (si, 0, bi)),
        out_shape=jax.ShapeDtypeStruct((S, D, Bx), jnp.float32),
    )
    return fmt(y3)


def kernel(x, token_embedding_table):
    Bx, S = x.shape
    V, D = token_embedding_table.shape
    B = Bx * S
    b_per_w = B // _NW
    chunk = 32
    n_outer = b_per_w // (chunk * _NBUF)
    flat = x.reshape(B).astype(jnp.int32)
    y = _sc_gather(token_embedding_table, flat, b_per_w, chunk, n_outer)
    y3 = y.reshape(Bx, S, D)
    out_t = _tc_format(y3, Bx, S, D, 128)
    return out_t.transpose(2, 0, 1)


# speedup vs baseline: 1.0478x; 1.0478x over previous
import functools

import jax
import jax.numpy as jnp
from jax import lax
from jax.experimental import pallas as pl
from jax.experimental.pallas import tpu as pltpu
from jax.experimental.pallas import tpu_sc as plsc

_NC = 2
_NS = 16
_NW = _NC * _NS
_NBUF = 2


@functools.partial(jax.jit, static_argnums=(2, 3, 4))
def _sc_gather(table, idx, b_per_w, chunk, n_outer):
    V, D = table.shape
    B = idx.shape[0]
    mesh = plsc.VectorSubcoreMesh(core_axis_name="c", subcore_axis_name="s")

    @functools.partial(
        pl.kernel,
        out_type=jax.ShapeDtypeStruct((B, D), jnp.float32),
        mesh=mesh,
        scratch_types=[
            pltpu.VMEM((b_per_w,), jnp.int32),
            [pltpu.VMEM((chunk, D), jnp.float32) for _ in range(_NBUF)],
            [pltpu.SemaphoreType.DMA for _ in range(_NBUF)],
            [pltpu.SemaphoreType.DMA for _ in range(_NBUF)],
        ],
    )
    def k(table_hbm, idx_hbm, out_hbm, idx_v, bufs, semg, semw):
        wid = lax.axis_index("s") * _NC + lax.axis_index("c")
        base = wid * b_per_w
        pltpu.sync_copy(idx_hbm.at[pl.ds(base, b_per_w)], idx_v)

        def gather_start(g, b):
            off = g * chunk
            for q in range(chunk // 16):
                vec = idx_v[pl.ds(off + q * 16, 16)]
                for r in range(16):
                    pltpu.make_async_copy(
                        table_hbm.at[pl.ds(vec[r], 1), :],
                        bufs[b].at[pl.ds(q * 16 + r, 1), :],
                        semg[b],
                    ).start()

        def gather_wait(b):
            pltpu.make_async_copy(
                table_hbm.at[pl.ds(0, chunk), :], bufs[b], semg[b]
            ).wait()

        def write_start(g, b):
            off = g * chunk
            pltpu.make_async_copy(
                bufs[b], out_hbm.at[pl.ds(base + off, chunk)], semw[b]
            ).start()

        def write_wait(b):
            pltpu.make_async_copy(
                bufs[b], out_hbm.at[pl.ds(base, chunk)], semw[b]
            ).wait()

        for b in range(_NBUF):
            gather_start(b, b)

        def body(j, _):
            for b in range(_NBUF):
                gather_wait(b)
                write_start(j * _NBUF + b, b)

            for b in range(_NBUF):
                write_wait(b)

                @pl.when(j < n_outer - 1)
                def _():
                    gather_start((j + 1) * _NBUF + b, b)

            return ()

        lax.fori_loop(0, n_outer, body, ())

    return k(table, idx)


def _fmt_body(y_ref, o_ref):
    x = y_ref[...]
    for s in range(y_ref.shape[1]):
        o_ref[s] = x[:, s, :].T


def _fmt_body_alias(y_ref, a_ref, o_ref):
    del a_ref
    _fmt_body(y_ref, o_ref)


@functools.partial(jax.jit, static_argnums=(2, 3, 4, 5, 6, 7))
def _tc_format_chunk(y3c, out_prev, c, K, Bx, S, D, bb):
    sb = 8
    bc = Bx // K
    out_map = lambda bi, si: (si, 0, c * (bc // bb) + bi)
    common = dict(
        grid=(bc // bb, S // sb),
        out_specs=pl.BlockSpec((sb, D, bb), out_map),
        out_shape=jax.ShapeDtypeStruct((S, D, Bx), jnp.float32),
    )
    y_spec = pl.BlockSpec((bb, sb, D), lambda bi, si: (bi, si, 0))
    if out_prev is None:
        fmt = pl.pallas_call(_fmt_body, in_specs=[y_spec], **common)
        return fmt(y3c)
    fmt = pl.pallas_call(
        _fmt_body_alias,
        in_specs=[y_spec, pl.BlockSpec(memory_space=pl.ANY)],
        input_output_aliases={1: 0},
        **common,
    )
    return fmt(y3c, out_prev)


def kernel(x, token_embedding_table):
    Bx, S = x.shape
    V, D = token_embedding_table.shape
    B = Bx * S
    K = 4
    bk = B // K
    b_per_w = bk // _NW
    chunk = 32
    n_outer = b_per_w // (chunk * _NBUF)
    flat = x.reshape(B).astype(jnp.int32)
    out_t = None
    for c in range(K):
        yc = _sc_gather(
            token_embedding_table,
            lax.dynamic_slice_in_dim(flat, c * bk, bk),
            b_per_w,
            chunk,
            n_outer,
        )
        y3c = yc.reshape(Bx // K, S, D)
        out_t = _tc_format_chunk(y3c, out_t, c, K, Bx, S, D, 128)
    return out_t.transpose(2, 0, 1)


# speedup vs baseline: 1.3283x; 1.2677x over previous
import functools

import jax
import jax.numpy as jnp
from jax import lax
from jax.experimental import pallas as pl
from jax.experimental.pallas import tpu as pltpu
from jax.experimental.pallas import tpu_sc as plsc

_NC = 2
_NS = 16
_NW = _NC * _NS
_NBUF = 2


@functools.partial(jax.jit, static_argnums=(2, 3, 4))
def _sc_gather(table, idx, b_per_w, chunk, n_outer):
    V, D = table.shape
    B = idx.shape[0]
    mesh = plsc.VectorSubcoreMesh(core_axis_name="c", subcore_axis_name="s")

    @functools.partial(
        pl.kernel,
        out_type=jax.ShapeDtypeStruct((B, D), jnp.float32),
        mesh=mesh,
        scratch_types=[
            pltpu.VMEM((b_per_w,), jnp.int32),
            pltpu.MemorySpace.VMEM_SHARED((V, D), jnp.float32),
            [pltpu.VMEM((chunk, D), jnp.float32) for _ in range(_NBUF)],
            [pltpu.SemaphoreType.DMA for _ in range(_NBUF)],
            [pltpu.SemaphoreType.DMA for _ in range(_NBUF)],
        ],
    )
    def k(table_hbm, idx_hbm, out_hbm, idx_v, tab_sp, bufs, semg, semw):
        wid = lax.axis_index("s") * _NC + lax.axis_index("c")
        base = wid * b_per_w
        @pl.when(lax.axis_index("s") == 0)
        def _():
            pltpu.sync_copy(table_hbm, tab_sp)

        pltpu.sync_copy(idx_hbm.at[pl.ds(base, b_per_w)], idx_v)
        plsc.subcore_barrier()

        def gather_start(g, b):
            off = g * chunk
            for q in range(chunk // 16):
                vec = idx_v[pl.ds(off + q * 16, 16)]
                for r in range(16):
                    pltpu.make_async_copy(
                        tab_sp.at[pl.ds(vec[r], 1), :],
                        bufs[b].at[pl.ds(q * 16 + r, 1), :],
                        semg[b],
                    ).start()

        def gather_wait(b):
            pltpu.make_async_copy(
                tab_sp.at[pl.ds(0, chunk), :], bufs[b], semg[b]
            ).wait()

        def write_start(g, b):
            off = g * chunk
            pltpu.make_async_copy(
                bufs[b], out_hbm.at[pl.ds(base + off, chunk)], semw[b]
            ).start()

        def write_wait(b):
            pltpu.make_async_copy(
                bufs[b], out_hbm.at[pl.ds(base, chunk)], semw[b]
            ).wait()

        for b in range(_NBUF):
            gather_start(b, b)

        def body(j, _):
            for b in range(_NBUF):
                gather_wait(b)
                write_start(j * _NBUF + b, b)

            for b in range(_NBUF):
                write_wait(b)

                @pl.when(j < n_outer - 1)
                def _():
                    gather_start((j + 1) * _NBUF + b, b)

            return ()

        lax.fori_loop(0, n_outer, body, ())

    return k(table, idx)


def _fmt_body(y_ref, o_ref):
    x = y_ref[...]
    for s in range(y_ref.shape[1]):
        o_ref[s] = x[:, s, :].T


def _fmt_body_alias(y_ref, a_ref, o_ref):
    del a_ref
    _fmt_body(y_ref, o_ref)


@functools.partial(jax.jit, static_argnums=(2, 3, 4, 5, 6, 7))
def _tc_format_chunk(y3c, out_prev, c, K, Bx, S, D, bb):
    sb = 8
    bc = Bx // K
    out_map = lambda bi, si: (si, 0, c * (bc // bb) + bi)
    common = dict(
        grid=(bc // bb, S // sb),
        out_specs=pl.BlockSpec((sb, D, bb), out_map),
        out_shape=jax.ShapeDtypeStruct((S, D, Bx), jnp.float32),
    )
    y_spec = pl.BlockSpec((bb, sb, D), lambda bi, si: (bi, si, 0))
    if out_prev is None:
        fmt = pl.pallas_call(_fmt_body, in_specs=[y_spec], **common)
        return fmt(y3c)
    fmt = pl.pallas_call(
        _fmt_body_alias,
        in_specs=[y_spec, pl.BlockSpec(memory_space=pl.ANY)],
        input_output_aliases={1: 0},
        **common,
    )
    return fmt(y3c, out_prev)


def kernel(x, token_embedding_table):
    Bx, S = x.shape
    V, D = token_embedding_table.shape
    B = Bx * S
    K = 4
    bk = B // K
    b_per_w = bk // _NW
    chunk = 32
    n_outer = b_per_w // (chunk * _NBUF)
    flat = x.reshape(B).astype(jnp.int32)
    out_t = None
    for c in range(K):
        yc = _sc_gather(
            token_embedding_table,
            lax.dynamic_slice_in_dim(flat, c * bk, bk),
            b_per_w,
            chunk,
            n_outer,
        )
        y3c = yc.reshape(Bx // K, S, D)
        out_t = _tc_format_chunk(y3c, out_t, c, K, Bx, S, D, 128)
    return out_t.transpose(2, 0, 1)
